# baseline (speedup 1.0000x reference)
import functools
import math

import jax
import jax.numpy as jnp
from jax import lax
from jax.experimental import pallas as pl
from jax.experimental.pallas import tpu as pltpu
from jax.experimental.pallas import tpu_sc as plsc

VOCAB = 1_000_000
DIM = 64
SCALE = math.sqrt(DIM)

NC = 2
NS = 16
NW = NC * NS

CHUNK = 128
B_TOTAL = 4096 * 200
N_CHUNKS = B_TOTAL // CHUNK
CHUNKS_PER_W = N_CHUNKS // NW
NB = 2


def _emb_body(table_h, idx_h, out_h,
              idx_v, rows_v, obuf_v,
              gsem0, gsem1, osem0, osem1):
    wid = lax.axis_index("s") * NC + lax.axis_index("c")
    base = wid * CHUNKS_PER_W

    pltpu.sync_copy(idx_h.at[pl.ds(base, CHUNKS_PER_W)], idx_v)

    gsems = (gsem0, gsem1)
    osems = (osem0, osem1)

    def issue_gather(j, b):
        pltpu.make_async_copy(
            table_h.at[idx_v.at[j]], rows_v.at[b], gsems[b]).start()

    def wait_gather(j, b):
        pltpu.make_async_copy(
            table_h.at[idx_v.at[j]], rows_v.at[b], gsems[b]).wait()

    def issue_out(j, b):
        pltpu.make_async_copy(
            obuf_v.at[b], out_h.at[base + j], osems[b]).start()

    def wait_out(j, b):
        pltpu.make_async_copy(
            obuf_v.at[b], out_h.at[base + j], osems[b]).wait()

    for b in range(NB):
        issue_gather(b, b)

    def chunk_step(t, carry):
        jj = t * NB
        for b in range(NB):
            j = jj + b
            wait_gather(j, b)

            @pl.when(j >= NB)
            def _wait_prev_out():
                wait_out(j - NB, b)

            def scale_row(r, c):
                for d in range(DIM // 16):
                    sl = pl.ds(d * 16, 16)
                    obuf_v[b, r, sl] = rows_v[b, r, sl] * jnp.float32(SCALE)
                return c

            lax.fori_loop(0, CHUNK, scale_row, 0, unroll=2)

            @pl.when(j + NB < CHUNKS_PER_W)
            def _issue_next_gather():
                issue_gather(j + NB, b)

            issue_out(j, b)
        return carry

    lax.fori_loop(0, CHUNKS_PER_W // NB, chunk_step, 0)

    for b in range(NB):
        wait_out(CHUNKS_PER_W - NB + b, b)


@jax.jit
def _emb_call(x_flat, table):
    mesh = plsc.VectorSubcoreMesh(core_axis_name="c", subcore_axis_name="s")
    kfn = pl.kernel(
        _emb_body,
        out_type=jax.ShapeDtypeStruct((N_CHUNKS, CHUNK, DIM), jnp.float32),
        mesh=mesh,
        compiler_params=pltpu.CompilerParams(use_tc_tiling_on_sc=False),
        scratch_types=[
            pltpu.VMEM((CHUNKS_PER_W, CHUNK), jnp.int32),
            pltpu.VMEM((NB, CHUNK, DIM), jnp.float32),
            pltpu.VMEM((NB, CHUNK, DIM), jnp.float32),
            pltpu.SemaphoreType.DMA,
            pltpu.SemaphoreType.DMA,
            pltpu.SemaphoreType.DMA,
            pltpu.SemaphoreType.DMA,
        ],
    )
    return kfn(table, x_flat)


def kernel(x, table):
    x_flat = x.reshape(N_CHUNKS, CHUNK).astype(jnp.int32)
    out = _emb_call(x_flat, table)
    return out.reshape(x.shape[0], x.shape[1], DIM)
